# Initial kernel scaffold; baseline (speedup 1.0000x reference)
import jax
import jax.numpy as jnp
from jax.experimental import pallas as pl


def kernel(scores, gumbel):
    raise NotImplementedError("write your pallas kernel here")


# speedup vs baseline: 1.0542x; 1.0542x over previous
import jax
import jax.numpy as jnp
from jax.experimental import pallas as pl
from jax.experimental.pallas import tpu as pltpu

R = 64
N = 1_000_000
C = 32768
GRID = (N + C - 1) // C


def _body(s_ref, g_ref, o_ref, m_ref, i_ref):
    j = pl.program_id(0)
    base = j * C
    v = s_ref[...] + g_ref[...]
    col = jax.lax.broadcasted_iota(jnp.int32, (R, C), 1)
    v = jnp.where(base + col < N, v, -jnp.inf)
    bm = jnp.max(v, axis=1, keepdims=True)
    bi = jnp.min(jnp.where(v == bm, col, C), axis=1, keepdims=True) + base

    @pl.when(j == 0)
    def _():
        m_ref[...] = bm
        i_ref[...] = bi

    @pl.when(j > 0)
    def _():
        upd = bm > m_ref[...]
        i_ref[...] = jnp.where(upd, bi, i_ref[...])
        m_ref[...] = jnp.where(upd, bm, m_ref[...])

    @pl.when(j == pl.num_programs(0) - 1)
    def _():
        o_ref[...] = i_ref[...]


def kernel(scores, gumbel):
    out = pl.pallas_call(
        _body,
        grid=(GRID,),
        in_specs=[
            pl.BlockSpec((R, C), lambda j: (0, j)),
            pl.BlockSpec((R, C), lambda j: (0, j)),
        ],
        out_specs=pl.BlockSpec((R, 1), lambda j: (0, 0)),
        out_shape=jax.ShapeDtypeStruct((R, 1), jnp.int32),
        scratch_shapes=[
            pltpu.VMEM((R, 1), jnp.float32),
            pltpu.VMEM((R, 1), jnp.int32),
        ],
        compiler_params=pltpu.CompilerParams(
            dimension_semantics=("arbitrary",),
        ),
    )(scores, gumbel)
    return out
